# Initial kernel scaffold; baseline (speedup 1.0000x reference)
import jax
import jax.numpy as jnp
from jax.experimental import pallas as pl


def kernel(sample, tables, W, b):
    raise NotImplementedError("write your pallas kernel here")


# speedup vs baseline: 1.2966x; 1.2966x over previous
import functools
import math

import jax
import jax.numpy as jnp
from jax import lax
from jax.experimental import pallas as pl
from jax.experimental.pallas import tpu as pltpu
from jax.experimental.pallas import tpu_sc as plsc

_N_FEATS = 26
_VOCAB = 100000
_D_EMBED = 32
_D_MODEL = 1024
_B, _L = 1024, 50
_NTOK = _B * _L
_R = _NTOK * _N_FEATS

_NW = 32
_ROWS_PER_TILE = _R // _NW
_SUB = 128
_NSUB = 5
_CHUNK = _SUB * _NSUB
_NCHUNK = _ROWS_PER_TILE // _CHUNK
assert _ROWS_PER_TILE % _CHUNK == 0


def _sc_gather_body(tables_hbm, samp_hbm, out_hbm, samp_v, idx_v, rows_v, gsem):
    nc = 2
    wid = lax.axis_index("s") * nc + lax.axis_index("c")
    base = wid * _ROWS_PER_TILE
    blk_base = wid * (_ROWS_PER_TILE // _SUB)

    def chunk(c, _):
        off = base + c * _CHUNK
        pltpu.sync_copy(samp_hbm.at[pl.ds(off, _CHUNK)], samp_v)
        for j in range(_NSUB):
            for k in range(_SUB // 16):
                p = j * _SUB + k * 16
                s = samp_v[pl.ds(p, 16)]
                r = lax.iota(jnp.int32, 16) + (off + p)
                f = lax.rem(r, _N_FEATS)
                idx_v[j, pl.ds(k * 16, 16)] = s + f * _VOCAB
        cps = [
            pltpu.async_copy(tables_hbm.at[idx_v.at[j]], rows_v.at[j], gsem)
            for j in range(_NSUB)
        ]
        for cp in cps:
            cp.wait()
        pltpu.sync_copy(rows_v, out_hbm.at[pl.ds(blk_base + c * _NSUB, _NSUB)])
        return ()

    lax.fori_loop(0, _NCHUNK, chunk, ())


@functools.cache
def _sc_gather():
    return pl.kernel(
        _sc_gather_body,
        out_type=jax.ShapeDtypeStruct((_R // _SUB, _SUB, _D_EMBED), jnp.float32),
        mesh=plsc.VectorSubcoreMesh(core_axis_name="c", subcore_axis_name="s"),
        scratch_types=[
            pltpu.VMEM((_CHUNK,), jnp.int32),
            pltpu.VMEM((_NSUB, _SUB), jnp.int32),
            pltpu.VMEM((_NSUB, _SUB, _D_EMBED), jnp.float32),
            pltpu.SemaphoreType.DMA,
        ],
        compiler_params=pltpu.CompilerParams(use_tc_tiling_on_sc=False),
    )


_TM = 256


def _proj_body(emb_ref, samp_ref, wt_ref, e_ref, b_ref, o_ref):
    mask = (samp_ref[...] != 0).astype(jnp.bfloat16)
    m832 = lax.dot_general(
        mask, e_ref[...], (((1,), (0,)), ((), ())),
        preferred_element_type=jnp.float32)
    me = (emb_ref[...] * m832).astype(jnp.bfloat16)
    acc = lax.dot_general(
        me, wt_ref[...], (((1,), (0,)), ((), ())),
        preferred_element_type=jnp.float32)
    o_ref[...] = (acc + b_ref[...]) * math.sqrt(float(_D_MODEL))


def _projection(embeds, samp_flat, wt_bf, e_bf, b_row):
    grid = (_NTOK // _TM,)
    return pl.pallas_call(
        _proj_body,
        grid=grid,
        in_specs=[
            pl.BlockSpec((_TM, _N_FEATS * _D_EMBED), lambda i: (i, 0)),
            pl.BlockSpec((_TM, _N_FEATS), lambda i: (i, 0)),
            pl.BlockSpec((_N_FEATS * _D_EMBED, _D_MODEL), lambda i: (0, 0)),
            pl.BlockSpec((_N_FEATS, _N_FEATS * _D_EMBED), lambda i: (0, 0)),
            pl.BlockSpec((1, _D_MODEL), lambda i: (0, 0)),
        ],
        out_specs=pl.BlockSpec((_TM, _D_MODEL), lambda i: (i, 0)),
        out_shape=jax.ShapeDtypeStruct((_NTOK, _D_MODEL), jnp.float32),
    )(embeds, samp_flat, wt_bf, e_bf, b_row)


def kernel(sample, tables, W, b):
    tables_flat = tables.reshape(_N_FEATS * _VOCAB, _D_EMBED)
    samp_flat = sample.reshape(_NTOK, _N_FEATS).astype(jnp.int32)
    samp_lin = samp_flat.reshape(_R)

    embeds = _sc_gather()(tables_flat, samp_lin)
    embeds = embeds.reshape(_NTOK, _N_FEATS * _D_EMBED)

    wt_bf = W.T.astype(jnp.bfloat16)
    e_bf = (jnp.arange(_N_FEATS * _D_EMBED)[None, :] // _D_EMBED
            == jnp.arange(_N_FEATS)[:, None]).astype(jnp.bfloat16)
    b_row = b.reshape(1, _D_MODEL)

    out = _projection(embeds, samp_flat, wt_bf, e_bf, b_row)
    return out.reshape(_B, _L, _D_MODEL)


# speedup vs baseline: 2.5736x; 1.9849x over previous
import functools
import math

import jax
import jax.numpy as jnp
from jax import lax
from jax.experimental import pallas as pl
from jax.experimental.pallas import tpu as pltpu
from jax.experimental.pallas import tpu_sc as plsc

_N_FEATS = 26
_VOCAB = 100000
_D_EMBED = 32
_D_MODEL = 1024
_B, _L = 1024, 50
_NTOK = _B * _L
_P = _N_FEATS * _D_EMBED

_NW = 32
_ROWS_PER_TILE = _P // _NW
_TCH = 5120
_NTCH = _NTOK // _TCH
assert _NTOK % _TCH == 0 and _TCH % 256 == 0


def _sc_gather_body(t_hbm, samp_hbm, out_hbm, vec_v, idx_v, out_v):
    wid = lax.axis_index("s") * 2 + lax.axis_index("c")
    p0 = wid * _ROWS_PER_TILE

    def prow(i, _):
        p = p0 + i
        f = p // _D_EMBED
        pltpu.sync_copy(t_hbm.at[p], vec_v)

        def tchunk(c, _):
            pltpu.sync_copy(samp_hbm.at[f, pl.ds(c * _TCH, _TCH)], idx_v)

            def g256(k, _):
                for u in range(16):
                    s = k * 256 + u * 16
                    iv = idx_v[pl.ds(s, 16)]
                    out_v[pl.ds(s, 16)] = plsc.load_gather(vec_v, [iv])
                return ()

            lax.fori_loop(0, _TCH // 256, g256, ())
            pltpu.sync_copy(out_v, out_hbm.at[p, pl.ds(c * _TCH, _TCH)])
            return ()

        lax.fori_loop(0, _NTCH, tchunk, ())
        return ()

    lax.fori_loop(0, _ROWS_PER_TILE, prow, ())


@functools.cache
def _sc_gather():
    return pl.kernel(
        _sc_gather_body,
        out_type=jax.ShapeDtypeStruct((_P, _NTOK), jnp.float32),
        mesh=plsc.VectorSubcoreMesh(core_axis_name="c", subcore_axis_name="s"),
        scratch_types=[
            pltpu.VMEM((_VOCAB,), jnp.float32),
            pltpu.VMEM((_TCH,), jnp.int32),
            pltpu.VMEM((_TCH,), jnp.float32),
        ],
        compiler_params=pltpu.CompilerParams(
            use_tc_tiling_on_sc=True, needs_layout_passes=False),
    )


_TM = 256


def _proj_body(embt_ref, samp_ref, wt_ref, et_ref, b_ref, o_ref):
    mask = (samp_ref[...] != 0).astype(jnp.bfloat16)
    m_t = lax.dot_general(
        et_ref[...], mask, (((1,), (1,)), ((), ())),
        preferred_element_type=jnp.float32)
    xm = (embt_ref[...] * m_t).astype(jnp.bfloat16)
    acc = lax.dot_general(
        xm, wt_ref[...], (((0,), (0,)), ((), ())),
        preferred_element_type=jnp.float32)
    o_ref[...] = (acc + b_ref[...]) * math.sqrt(float(_D_MODEL))


def _projection(embt, samp_u, wt_bf, et_bf, b_row):
    return pl.pallas_call(
        _proj_body,
        grid=(_NTOK // _TM,),
        in_specs=[
            pl.BlockSpec((_P, _TM), lambda i: (0, i)),
            pl.BlockSpec((_TM, _N_FEATS), lambda i: (i, 0)),
            pl.BlockSpec((_P, _D_MODEL), lambda i: (0, 0)),
            pl.BlockSpec((_P, _N_FEATS), lambda i: (0, 0)),
            pl.BlockSpec((1, _D_MODEL), lambda i: (0, 0)),
        ],
        out_specs=pl.BlockSpec((_TM, _D_MODEL), lambda i: (i, 0)),
        out_shape=jax.ShapeDtypeStruct((_NTOK, _D_MODEL), jnp.float32),
    )(embt, samp_u, wt_bf, et_bf, b_row)


def kernel(sample, tables, W, b):
    t_flat = tables.transpose(0, 2, 1).reshape(_P, _VOCAB)
    samp_f = sample.transpose(2, 1, 0).reshape(_N_FEATS, _NTOK).astype(jnp.int32)
    samp_u = sample.transpose(1, 0, 2).reshape(_NTOK, _N_FEATS).astype(jnp.int32)

    embt = _sc_gather()(t_flat, samp_f)

    wt_bf = W.T.astype(jnp.bfloat16)
    et_bf = (jnp.arange(_P)[:, None] // _D_EMBED
             == jnp.arange(_N_FEATS)[None, :]).astype(jnp.bfloat16)
    b_row = b.reshape(1, _D_MODEL)

    out = _projection(embt, samp_u, wt_bf, et_bf, b_row)
    return out.reshape(_L, _B, _D_MODEL).transpose(1, 0, 2)


# speedup vs baseline: 3.1089x; 1.2080x over previous
import functools
import math

import jax
import jax.numpy as jnp
from jax import lax
from jax.experimental import pallas as pl
from jax.experimental.pallas import tpu as pltpu
from jax.experimental.pallas import tpu_sc as plsc

_N_FEATS = 26
_VOCAB = 100000
_D_EMBED = 32
_D_MODEL = 1024
_B, _L = 1024, 50
_NTOK = _B * _L
_P = _N_FEATS * _D_EMBED

_NW = 32
_ROWS_PER_TILE = _P // _NW
_TCH = 5120
_NTCH = _NTOK // _TCH
assert _NTOK % _TCH == 0 and _TCH % 256 == 0


def _sc_gather_body(t_hbm, samp_hbm, out_hbm, vec_v, idx_v, out_v,
                    isems, osems):
    wid = lax.axis_index("s") * 2 + lax.axis_index("c")
    p0 = wid * _ROWS_PER_TILE

    def prow(i, _):
        p = p0 + i
        f = p // _D_EMBED
        pltpu.async_copy(samp_hbm.at[f, pl.ds(0, _TCH)], idx_v.at[0], isems[0])
        pltpu.sync_copy(t_hbm.at[p], vec_v)

        @pl.when(i > 0)
        def _():
            for b in range(2):
                pltpu.make_async_copy(
                    out_v.at[b], out_hbm.at[p, pl.ds(0, _TCH)], osems[b]).wait()

        for c in range(_NTCH):
            b = c % 2
            if c + 1 < _NTCH:
                pltpu.async_copy(samp_hbm.at[f, pl.ds((c + 1) * _TCH, _TCH)],
                                 idx_v.at[1 - b], isems[1 - b])
            pltpu.make_async_copy(
                samp_hbm.at[f, pl.ds(c * _TCH, _TCH)], idx_v.at[b],
                isems[b]).wait()
            if c >= 2:
                pltpu.make_async_copy(
                    out_v.at[b], out_hbm.at[p, pl.ds((c - 2) * _TCH, _TCH)],
                    osems[b]).wait()

            def g256(k, _, b=b):
                for u in range(16):
                    s = k * 256 + u * 16
                    iv = idx_v[b, pl.ds(s, 16)]
                    out_v[b, pl.ds(s, 16)] = plsc.load_gather(vec_v, [iv])
                return ()

            lax.fori_loop(0, _TCH // 256, g256, ())
            pltpu.async_copy(out_v.at[b], out_hbm.at[p, pl.ds(c * _TCH, _TCH)],
                             osems[b])
        return ()

    lax.fori_loop(0, _ROWS_PER_TILE, prow, ())
    for b in range(2):
        pltpu.make_async_copy(
            out_v.at[b], out_hbm.at[0, pl.ds(0, _TCH)], osems[b]).wait()


@functools.cache
def _sc_gather():
    return pl.kernel(
        _sc_gather_body,
        out_type=jax.ShapeDtypeStruct((_P, _NTOK), jnp.float32),
        mesh=plsc.VectorSubcoreMesh(core_axis_name="c", subcore_axis_name="s"),
        scratch_types=[
            pltpu.VMEM((_VOCAB,), jnp.float32),
            pltpu.VMEM((2, _TCH), jnp.int32),
            pltpu.VMEM((2, _TCH), jnp.float32),
            [pltpu.SemaphoreType.DMA, pltpu.SemaphoreType.DMA],
            [pltpu.SemaphoreType.DMA, pltpu.SemaphoreType.DMA],
        ],
        compiler_params=pltpu.CompilerParams(
            use_tc_tiling_on_sc=True, needs_layout_passes=False),
    )


_TM = 256


def _proj_body(embt_ref, samp_ref, wt_ref, et_ref, b_ref, o_ref):
    mask = (samp_ref[...] != 0).astype(jnp.bfloat16)
    m_t = lax.dot_general(
        et_ref[...], mask, (((1,), (1,)), ((), ())),
        preferred_element_type=jnp.float32)
    xm = (embt_ref[...] * m_t).astype(jnp.bfloat16)
    acc = lax.dot_general(
        xm, wt_ref[...], (((0,), (0,)), ((), ())),
        preferred_element_type=jnp.float32)
    o_ref[...] = (acc + b_ref[...]) * math.sqrt(float(_D_MODEL))


def _projection(embt, samp_u, wt_bf, et_bf, b_row):
    return pl.pallas_call(
        _proj_body,
        grid=(_NTOK // _TM,),
        in_specs=[
            pl.BlockSpec((_P, _TM), lambda i: (0, i)),
            pl.BlockSpec((_TM, _N_FEATS), lambda i: (i, 0)),
            pl.BlockSpec((_P, _D_MODEL), lambda i: (0, 0)),
            pl.BlockSpec((_P, _N_FEATS), lambda i: (0, 0)),
            pl.BlockSpec((1, _D_MODEL), lambda i: (0, 0)),
        ],
        out_specs=pl.BlockSpec((_TM, _D_MODEL), lambda i: (i, 0)),
        out_shape=jax.ShapeDtypeStruct((_NTOK, _D_MODEL), jnp.float32),
    )(embt, samp_u, wt_bf, et_bf, b_row)


def kernel(sample, tables, W, b):
    t_flat = tables.transpose(0, 2, 1).reshape(_P, _VOCAB)
    samp_f = sample.transpose(2, 1, 0).reshape(_N_FEATS, _NTOK).astype(jnp.int32)
    samp_u = sample.transpose(1, 0, 2).reshape(_NTOK, _N_FEATS).astype(jnp.int32)

    embt = _sc_gather()(t_flat, samp_f)

    wt_bf = W.T.astype(jnp.bfloat16)
    et_bf = (jnp.arange(_P)[:, None] // _D_EMBED
             == jnp.arange(_N_FEATS)[None, :]).astype(jnp.bfloat16)
    b_row = b.reshape(1, _D_MODEL)

    out = _projection(embt, samp_u, wt_bf, et_bf, b_row)
    return out.reshape(_L, _B, _D_MODEL).transpose(1, 0, 2)


# speedup vs baseline: 4.1871x; 1.3468x over previous
import functools
import math

import jax
import jax.numpy as jnp
from jax import lax
from jax.experimental import pallas as pl
from jax.experimental.pallas import tpu as pltpu
from jax.experimental.pallas import tpu_sc as plsc

_N_FEATS = 26
_VOCAB = 100000
_D_EMBED = 32
_D_MODEL = 1024
_B, _L = 1024, 50
_NTOK = _B * _L
_P = _N_FEATS * _D_EMBED

_NW = 32
_ROWS_PER_TILE = _P // _NW
_TCH = 5120
_NTCH = _NTOK // _TCH
assert _NTOK % _TCH == 0 and _TCH % 256 == 0


def _sc_gather_body(t_hbm, samp_hbm, out_hbm, vec_v, idx_v, out_v,
                    isems, osems):
    wid = lax.axis_index("s") * 2 + lax.axis_index("c")
    p0 = wid * _ROWS_PER_TILE

    def prow(i, _):
        p = p0 + i
        f = p // _D_EMBED
        pltpu.async_copy(samp_hbm.at[f, pl.ds(0, _TCH)], idx_v.at[0], isems[0])
        pltpu.sync_copy(t_hbm.at[p], vec_v)

        @pl.when(i > 0)
        def _():
            for b in range(2):
                pltpu.make_async_copy(
                    out_v.at[b], out_hbm.at[p, pl.ds(0, _TCH)], osems[b]).wait()

        for c in range(_NTCH):
            b = c % 2
            if c + 1 < _NTCH:
                pltpu.async_copy(samp_hbm.at[f, pl.ds((c + 1) * _TCH, _TCH)],
                                 idx_v.at[1 - b], isems[1 - b])
            pltpu.make_async_copy(
                samp_hbm.at[f, pl.ds(c * _TCH, _TCH)], idx_v.at[b],
                isems[b]).wait()
            if c >= 2:
                pltpu.make_async_copy(
                    out_v.at[b], out_hbm.at[p, pl.ds((c - 2) * _TCH, _TCH)],
                    osems[b]).wait()

            @plsc.parallel_loop(0, _TCH, 16, unroll=8)
            def _gather(s, b=b):
                iv = idx_v[b, pl.ds(s, 16)]
                out_v[b, pl.ds(s, 16)] = plsc.load_gather(vec_v, [iv])
            pltpu.async_copy(out_v.at[b], out_hbm.at[p, pl.ds(c * _TCH, _TCH)],
                             osems[b])
        return ()

    lax.fori_loop(0, _ROWS_PER_TILE, prow, ())
    for b in range(2):
        pltpu.make_async_copy(
            out_v.at[b], out_hbm.at[0, pl.ds(0, _TCH)], osems[b]).wait()


@functools.cache
def _sc_gather():
    return pl.kernel(
        _sc_gather_body,
        out_type=jax.ShapeDtypeStruct((_P, _NTOK), jnp.float32),
        mesh=plsc.VectorSubcoreMesh(core_axis_name="c", subcore_axis_name="s"),
        scratch_types=[
            pltpu.VMEM((_VOCAB,), jnp.float32),
            pltpu.VMEM((2, _TCH), jnp.int32),
            pltpu.VMEM((2, _TCH), jnp.float32),
            [pltpu.SemaphoreType.DMA, pltpu.SemaphoreType.DMA],
            [pltpu.SemaphoreType.DMA, pltpu.SemaphoreType.DMA],
        ],
        compiler_params=pltpu.CompilerParams(
            use_tc_tiling_on_sc=True, needs_layout_passes=False),
    )


_TM = 256


def _proj_body(embt_ref, samp_ref, wt_ref, et_ref, b_ref, o_ref):
    mask = (samp_ref[...] != 0).astype(jnp.bfloat16)
    m_t = lax.dot_general(
        et_ref[...], mask, (((1,), (1,)), ((), ())),
        preferred_element_type=jnp.float32)
    xm = (embt_ref[...] * m_t).astype(jnp.bfloat16)
    acc = lax.dot_general(
        xm, wt_ref[...], (((0,), (0,)), ((), ())),
        preferred_element_type=jnp.float32)
    o_ref[...] = (acc + b_ref[...]) * math.sqrt(float(_D_MODEL))


def _projection(embt, samp_u, wt_bf, et_bf, b_row):
    return pl.pallas_call(
        _proj_body,
        grid=(_NTOK // _TM,),
        in_specs=[
            pl.BlockSpec((_P, _TM), lambda i: (0, i)),
            pl.BlockSpec((_TM, _N_FEATS), lambda i: (i, 0)),
            pl.BlockSpec((_P, _D_MODEL), lambda i: (0, 0)),
            pl.BlockSpec((_P, _N_FEATS), lambda i: (0, 0)),
            pl.BlockSpec((1, _D_MODEL), lambda i: (0, 0)),
        ],
        out_specs=pl.BlockSpec((_TM, _D_MODEL), lambda i: (i, 0)),
        out_shape=jax.ShapeDtypeStruct((_NTOK, _D_MODEL), jnp.float32),
    )(embt, samp_u, wt_bf, et_bf, b_row)


def kernel(sample, tables, W, b):
    t_flat = tables.transpose(0, 2, 1).reshape(_P, _VOCAB)
    samp_f = sample.transpose(2, 1, 0).reshape(_N_FEATS, _NTOK).astype(jnp.int32)
    samp_u = sample.transpose(1, 0, 2).reshape(_NTOK, _N_FEATS).astype(jnp.int32)

    embt = _sc_gather()(t_flat, samp_f)

    wt_bf = W.T.astype(jnp.bfloat16)
    et_bf = (jnp.arange(_P)[:, None] // _D_EMBED
             == jnp.arange(_N_FEATS)[None, :]).astype(jnp.bfloat16)
    b_row = b.reshape(1, _D_MODEL)

    out = _projection(embt, samp_u, wt_bf, et_bf, b_row)
    return out.reshape(_L, _B, _D_MODEL).transpose(1, 0, 2)


# speedup vs baseline: 4.7316x; 1.1300x over previous
import functools
import math

import jax
import jax.numpy as jnp
from jax import lax
from jax.experimental import pallas as pl
from jax.experimental.pallas import tpu as pltpu
from jax.experimental.pallas import tpu_sc as plsc

_N_FEATS = 26
_VOCAB = 100000
_D_EMBED = 32
_D_MODEL = 1024
_B, _L = 1024, 50
_NTOK = _B * _L
_P = _N_FEATS * _D_EMBED

_NW = 32
_ROWS_PER_TILE = _P // _NW
_TCH = 5120
_NTCH = _NTOK // _TCH
assert _NTOK % _TCH == 0 and _TCH % 256 == 0


def _sc_gather_body(t_hbm, samp_hbm, out_hbm, vec_v, idx_v, out_v,
                    isems, osems):
    wid = lax.axis_index("s") * 2 + lax.axis_index("c")
    p0 = wid * _ROWS_PER_TILE

    def prow(i, _):
        p = p0 + i
        f = p // _D_EMBED
        pltpu.async_copy(samp_hbm.at[f, pl.ds(0, _TCH)], idx_v.at[0], isems[0])
        pltpu.sync_copy(t_hbm.at[p], vec_v)

        @pl.when(i > 0)
        def _():
            for b in range(2):
                pltpu.make_async_copy(
                    out_v.at[b], out_hbm.at[p, pl.ds(0, _TCH)], osems[b]).wait()

        for c in range(_NTCH):
            b = c % 2
            if c + 1 < _NTCH:
                pltpu.async_copy(samp_hbm.at[f, pl.ds((c + 1) * _TCH, _TCH)],
                                 idx_v.at[1 - b], isems[1 - b])
            pltpu.make_async_copy(
                samp_hbm.at[f, pl.ds(c * _TCH, _TCH)], idx_v.at[b],
                isems[b]).wait()
            if c >= 2:
                pltpu.make_async_copy(
                    out_v.at[b], out_hbm.at[p, pl.ds((c - 2) * _TCH, _TCH)],
                    osems[b]).wait()

            @plsc.parallel_loop(0, _TCH, 16, unroll=16)
            def _gather(s, b=b):
                iv = idx_v[b, pl.ds(s, 16)]
                out_v[b, pl.ds(s, 16)] = plsc.load_gather(vec_v, [iv])
            pltpu.async_copy(out_v.at[b], out_hbm.at[p, pl.ds(c * _TCH, _TCH)],
                             osems[b])
        return ()

    lax.fori_loop(0, _ROWS_PER_TILE, prow, ())
    for b in range(2):
        pltpu.make_async_copy(
            out_v.at[b], out_hbm.at[0, pl.ds(0, _TCH)], osems[b]).wait()


@functools.cache
def _sc_gather():
    return pl.kernel(
        _sc_gather_body,
        out_type=jax.ShapeDtypeStruct((_P, _NTOK), jnp.float32),
        mesh=plsc.VectorSubcoreMesh(core_axis_name="c", subcore_axis_name="s"),
        scratch_types=[
            pltpu.VMEM((_VOCAB,), jnp.float32),
            pltpu.VMEM((2, _TCH), jnp.int32),
            pltpu.VMEM((2, _TCH), jnp.float32),
            [pltpu.SemaphoreType.DMA, pltpu.SemaphoreType.DMA],
            [pltpu.SemaphoreType.DMA, pltpu.SemaphoreType.DMA],
        ],
        compiler_params=pltpu.CompilerParams(
            use_tc_tiling_on_sc=True, needs_layout_passes=False),
    )


_TM = 512


def _proj_body(embt_ref, samp_ref, wt_ref, et_ref, b_ref, o_ref):
    mask = (samp_ref[...] != 0).astype(jnp.bfloat16)
    m_t = lax.dot_general(
        et_ref[...], mask, (((1,), (1,)), ((), ())),
        preferred_element_type=jnp.float32)
    xm = (embt_ref[...] * m_t).astype(jnp.bfloat16)
    acc = lax.dot_general(
        xm, wt_ref[...], (((0,), (0,)), ((), ())),
        preferred_element_type=jnp.float32)
    o_ref[...] = (acc + b_ref[...]) * math.sqrt(float(_D_MODEL))


def _projection(embt, samp_u, wt_bf, et_bf, b_row):
    return pl.pallas_call(
        _proj_body,
        grid=(_NTOK // _TM,),
        in_specs=[
            pl.BlockSpec((_P, _TM), lambda i: (0, i)),
            pl.BlockSpec((_TM, _N_FEATS), lambda i: (i, 0)),
            pl.BlockSpec((_P, _D_MODEL), lambda i: (0, 0)),
            pl.BlockSpec((_P, _N_FEATS), lambda i: (0, 0)),
            pl.BlockSpec((1, _D_MODEL), lambda i: (0, 0)),
        ],
        out_specs=pl.BlockSpec((_TM, _D_MODEL), lambda i: (i, 0)),
        out_shape=jax.ShapeDtypeStruct((_NTOK, _D_MODEL), jnp.float32),
    )(embt, samp_u, wt_bf, et_bf, b_row)


def kernel(sample, tables, W, b):
    t_flat = tables.transpose(0, 2, 1).reshape(_P, _VOCAB)
    samp_f = sample.transpose(2, 1, 0).reshape(_N_FEATS, _NTOK).astype(jnp.int32)
    samp_u = sample.transpose(1, 0, 2).reshape(_NTOK, _N_FEATS).astype(jnp.int32)

    embt = _sc_gather()(t_flat, samp_f)

    wt_bf = W.T.astype(jnp.bfloat16)
    et_bf = (jnp.arange(_P)[:, None] // _D_EMBED
             == jnp.arange(_N_FEATS)[None, :]).astype(jnp.bfloat16)
    b_row = b.reshape(1, _D_MODEL)

    out = _projection(embt, samp_u, wt_bf, et_bf, b_row)
    return out.reshape(_L, _B, _D_MODEL).transpose(1, 0, 2)


# speedup vs baseline: 5.1170x; 1.0815x over previous
import functools
import math

import jax
import jax.numpy as jnp
from jax import lax
from jax.experimental import pallas as pl
from jax.experimental.pallas import tpu as pltpu
from jax.experimental.pallas import tpu_sc as plsc

_N_FEATS = 26
_VOCAB = 100000
_D_EMBED = 32
_D_MODEL = 1024
_B, _L = 1024, 50
_NTOK = _B * _L
_P = _N_FEATS * _D_EMBED

_NW = 32
_ROWS_PER_TILE = _P // _NW
_TCH = 6400
_NTCH = _NTOK // _TCH
assert _NTOK % _TCH == 0 and _TCH % 256 == 0


def _sc_gather_body(t_hbm, samp_hbm, out_hbm, vec_v, idx_v, out_v,
                    isems, osems):
    wid = lax.axis_index("s") * 2 + lax.axis_index("c")
    p0 = wid * _ROWS_PER_TILE

    def prow(i, _):
        p = p0 + i
        f = p // _D_EMBED
        pltpu.async_copy(samp_hbm.at[f, pl.ds(0, _TCH)], idx_v.at[0], isems[0])
        pltpu.sync_copy(t_hbm.at[p], vec_v)

        @pl.when(i > 0)
        def _():
            for b in range(2):
                pltpu.make_async_copy(
                    out_v.at[b], out_hbm.at[p, pl.ds(0, _TCH)], osems[b]).wait()

        for c in range(_NTCH):
            b = c % 2
            if c + 1 < _NTCH:
                pltpu.async_copy(samp_hbm.at[f, pl.ds((c + 1) * _TCH, _TCH)],
                                 idx_v.at[1 - b], isems[1 - b])
            pltpu.make_async_copy(
                samp_hbm.at[f, pl.ds(c * _TCH, _TCH)], idx_v.at[b],
                isems[b]).wait()
            if c >= 2:
                pltpu.make_async_copy(
                    out_v.at[b], out_hbm.at[p, pl.ds((c - 2) * _TCH, _TCH)],
                    osems[b]).wait()

            @plsc.parallel_loop(0, _TCH, 16, unroll=16)
            def _gather(s, b=b):
                iv = idx_v[b, pl.ds(s, 16)]
                out_v[b, pl.ds(s, 16)] = plsc.load_gather(vec_v, [iv])
            pltpu.async_copy(out_v.at[b], out_hbm.at[p, pl.ds(c * _TCH, _TCH)],
                             osems[b])
        return ()

    lax.fori_loop(0, _ROWS_PER_TILE, prow, ())
    for b in range(2):
        pltpu.make_async_copy(
            out_v.at[b], out_hbm.at[0, pl.ds(0, _TCH)], osems[b]).wait()


@functools.cache
def _sc_gather():
    return pl.kernel(
        _sc_gather_body,
        out_type=jax.ShapeDtypeStruct((_P, _NTOK), jnp.float32),
        mesh=plsc.VectorSubcoreMesh(core_axis_name="c", subcore_axis_name="s"),
        scratch_types=[
            pltpu.VMEM((_VOCAB,), jnp.float32),
            pltpu.VMEM((2, _TCH), jnp.int32),
            pltpu.VMEM((2, _TCH), jnp.float32),
            [pltpu.SemaphoreType.DMA, pltpu.SemaphoreType.DMA],
            [pltpu.SemaphoreType.DMA, pltpu.SemaphoreType.DMA],
        ],
        compiler_params=pltpu.CompilerParams(
            use_tc_tiling_on_sc=True, needs_layout_passes=False),
    )


_TM = 1024


def _proj_body(embt_ref, samp_ref, wt_ref, et_ref, b_ref, o_ref):
    mask_t = (samp_ref[...] != 0).astype(jnp.bfloat16)
    m_t = lax.dot_general(
        et_ref[...], mask_t, (((1,), (0,)), ((), ())),
        preferred_element_type=jnp.float32)
    xm = (embt_ref[...] * m_t).astype(jnp.bfloat16)
    acc = lax.dot_general(
        xm, wt_ref[...], (((0,), (0,)), ((), ())),
        preferred_element_type=jnp.float32)
    o_ref[...] = (acc + b_ref[...]) * math.sqrt(float(_D_MODEL))


def _projection(embt, samp_u, wt_bf, et_bf, b_row):
    return pl.pallas_call(
        _proj_body,
        grid=(_NTOK // _TM,),
        in_specs=[
            pl.BlockSpec((_P, _TM), lambda i: (0, i)),
            pl.BlockSpec((_N_FEATS, _TM), lambda i: (0, i)),
            pl.BlockSpec((_P, _D_MODEL), lambda i: (0, 0)),
            pl.BlockSpec((_P, _N_FEATS), lambda i: (0, 0)),
            pl.BlockSpec((1, _D_MODEL), lambda i: (0, 0)),
        ],
        out_specs=pl.BlockSpec((_TM, _D_MODEL), lambda i: (i, 0)),
        out_shape=jax.ShapeDtypeStruct((_NTOK, _D_MODEL), jnp.float32),
    )(embt, samp_u, wt_bf, et_bf, b_row)


def kernel(sample, tables, W, b):
    t_flat = tables.transpose(0, 2, 1).reshape(_P, _VOCAB)
    samp_f = sample.transpose(2, 1, 0).reshape(_N_FEATS, _NTOK).astype(jnp.int32)

    embt = _sc_gather()(t_flat, samp_f)

    wt_bf = W.T.astype(jnp.bfloat16)
    et_bf = (jnp.arange(_P)[:, None] // _D_EMBED
             == jnp.arange(_N_FEATS)[None, :]).astype(jnp.bfloat16)
    b_row = b.reshape(1, _D_MODEL)

    out = _projection(embt, samp_f, wt_bf, et_bf, b_row)
    return out.reshape(_L, _B, _D_MODEL).transpose(1, 0, 2)


# speedup vs baseline: 5.1208x; 1.0007x over previous
import functools
import math

import jax
import jax.numpy as jnp
from jax import lax
from jax.experimental import pallas as pl
from jax.experimental.pallas import tpu as pltpu
from jax.experimental.pallas import tpu_sc as plsc

_N_FEATS = 26
_VOCAB = 100000
_D_EMBED = 32
_D_MODEL = 1024
_B, _L = 1024, 50
_NTOK = _B * _L
_P = _N_FEATS * _D_EMBED

_NW = 32
_ROWS_PER_TILE = _P // _NW
_TCH = 6400
_NTCH = _NTOK // _TCH
assert _NTOK % _TCH == 0 and _TCH % 256 == 0


def _sc_gather_body(t_hbm, samp_hbm, out_hbm, vec_v, idx_v, out_v,
                    isems, osems):
    wid = lax.axis_index("s") * 2 + lax.axis_index("c")
    p0 = wid * _ROWS_PER_TILE

    def prow(i, _):
        p = p0 + i
        f = p // _D_EMBED
        pltpu.async_copy(samp_hbm.at[f, pl.ds(0, _TCH)], idx_v.at[0], isems[0])
        pltpu.sync_copy(t_hbm.at[p], vec_v)

        @pl.when(i > 0)
        def _():
            for b in range(2):
                pltpu.make_async_copy(
                    out_v.at[b], out_hbm.at[p, pl.ds(0, _TCH)], osems[b]).wait()

        for c in range(_NTCH):
            b = c % 2
            if c + 1 < _NTCH:
                pltpu.async_copy(samp_hbm.at[f, pl.ds((c + 1) * _TCH, _TCH)],
                                 idx_v.at[1 - b], isems[1 - b])
            pltpu.make_async_copy(
                samp_hbm.at[f, pl.ds(c * _TCH, _TCH)], idx_v.at[b],
                isems[b]).wait()
            if c >= 2:
                pltpu.make_async_copy(
                    out_v.at[b], out_hbm.at[p, pl.ds((c - 2) * _TCH, _TCH)],
                    osems[b]).wait()

            @plsc.parallel_loop(0, _TCH, 16, unroll=16)
            def _gather(s, b=b):
                iv = idx_v[b, pl.ds(s, 16)]
                out_v[b, pl.ds(s, 16)] = plsc.load_gather(vec_v, [iv])
            pltpu.async_copy(out_v.at[b], out_hbm.at[p, pl.ds(c * _TCH, _TCH)],
                             osems[b])
        return ()

    lax.fori_loop(0, _ROWS_PER_TILE, prow, ())
    for b in range(2):
        pltpu.make_async_copy(
            out_v.at[b], out_hbm.at[0, pl.ds(0, _TCH)], osems[b]).wait()


@functools.cache
def _sc_gather():
    return pl.kernel(
        _sc_gather_body,
        out_type=jax.ShapeDtypeStruct((_P, _NTOK), jnp.float32),
        mesh=plsc.VectorSubcoreMesh(core_axis_name="c", subcore_axis_name="s"),
        scratch_types=[
            pltpu.VMEM((_VOCAB,), jnp.float32),
            pltpu.VMEM((2, _TCH), jnp.int32),
            pltpu.VMEM((2, _TCH), jnp.float32),
            [pltpu.SemaphoreType.DMA, pltpu.SemaphoreType.DMA],
            [pltpu.SemaphoreType.DMA, pltpu.SemaphoreType.DMA],
        ],
        compiler_params=pltpu.CompilerParams(
            use_tc_tiling_on_sc=True, needs_layout_passes=False),
    )


_TM = 1024


def _proj_body(embt_ref, samp_ref, wt_ref, et_ref, b_ref, o_ref):
    mask_t = (samp_ref[...] != 0).astype(jnp.bfloat16)
    m_t = lax.dot_general(
        et_ref[...], mask_t, (((1,), (0,)), ((), ())),
        preferred_element_type=jnp.float32)
    xm = (embt_ref[...] * m_t).astype(jnp.bfloat16)
    acc = lax.dot_general(
        xm, wt_ref[...], (((0,), (0,)), ((), ())),
        preferred_element_type=jnp.float32)
    o_ref[...] = acc + b_ref[...]


def _projection(embt, samp_u, wt_bf, et_bf, b_row):
    return pl.pallas_call(
        _proj_body,
        grid=(_NTOK // _TM,),
        in_specs=[
            pl.BlockSpec((_P, _TM), lambda i: (0, i)),
            pl.BlockSpec((_N_FEATS, _TM), lambda i: (0, i)),
            pl.BlockSpec((_P, _D_MODEL), lambda i: (0, 0)),
            pl.BlockSpec((_P, _N_FEATS), lambda i: (0, 0)),
            pl.BlockSpec((1, _D_MODEL), lambda i: (0, 0)),
        ],
        out_specs=pl.BlockSpec((_TM, _D_MODEL), lambda i: (i, 0)),
        out_shape=jax.ShapeDtypeStruct((_NTOK, _D_MODEL), jnp.float32),
    )(embt, samp_u, wt_bf, et_bf, b_row)


def kernel(sample, tables, W, b):
    t_flat = tables.transpose(0, 2, 1).reshape(_P, _VOCAB)
    samp_f = sample.transpose(2, 1, 0).reshape(_N_FEATS, _NTOK).astype(jnp.int32)

    embt = _sc_gather()(t_flat, samp_f)

    scale = math.sqrt(float(_D_MODEL))
    wt_bf = (W.T * scale).astype(jnp.bfloat16)
    et_bf = (jnp.arange(_P)[:, None] // _D_EMBED
             == jnp.arange(_N_FEATS)[None, :]).astype(jnp.bfloat16)
    b_row = (b * scale).reshape(1, _D_MODEL)

    out = _projection(embt, samp_f, wt_bf, et_bf, b_row)
    return out.reshape(_L, _B, _D_MODEL).transpose(1, 0, 2)
